# baseline (speedup 1.0000x reference)
import numpy as np
import jax
import jax.numpy as jnp
from jax import lax
from jax.experimental import pallas as pl
from jax.experimental.pallas import tpu as pltpu

_C = 128
_NID = 6


def _build_consts():
    np.random.seed(0)
    m = np.zeros((_C, _NID), dtype=np.float32)
    m[:, 1:_NID] = np.random.rand(_C, _NID - 1)
    colT = m.T

    X = np.zeros(_NID, dtype=np.uint32)
    for i in range(_NID):
        for y in range(32):
            if ((2 * y) >> i) & 1:
                X[i] |= np.uint32(1) << np.uint32(y)

    M = np.zeros(_NID, dtype=np.uint32)
    for j in range(1, _NID):
        for c in range(_C):
            pat = 0
            for i in range(_NID):
                if colT[i, c] >= colT[j, c]:
                    pat |= 1 << i
            M[j] |= np.uint32(1) << np.uint32(pat // 2)

    return ([int(v) for v in X.view(np.int32)],
            [int(v) for v in M.view(np.int32)])


_X32, _M32 = _build_consts()


def _select6(idx, consts):
    out = jnp.full(idx.shape, consts[0], dtype=jnp.int32)
    for i in range(1, _NID):
        out = jnp.where(idx == i, jnp.int32(consts[i]), out)
    return out


def _trust_body(c_ref, pred_ref, out_ref):
    nt = c_ref.shape[0]
    shp = c_ref.shape[1:]
    zero = jnp.zeros(shp, dtype=jnp.int32)
    neg1 = jnp.full(shp, -1, dtype=jnp.int32)

    def step(t, g):
        c = c_ref[t]
        s = c < -2.5
        f = c > 7.5
        idt = jnp.where(s, c + 8.0, jnp.where(f, c - 8.0, c))
        x = jnp.full(shp, _X32[0], dtype=jnp.int32)
        for i in range(1, _NID):
            x = jnp.where(idt == float(i), jnp.int32(_X32[i]), x)
        return (g | jnp.where(s, x, zero)) & jnp.where(f, x, neg1)

    g = lax.fori_loop(0, nt, step, zero, unroll=True)

    m = _select6(pred_ref[...], _M32)
    out_ref[...] = ((g & m) == m).astype(jnp.float32)


def kernel(inptasksobs, inptasksperf, inptaskspred, num_obs_tasks, tasksobsids, taskspredids):
    nt = tasksobsids.shape[0]
    nb = tasksobsids.shape[1]
    lanes = 128
    rows = nb // lanes

    c = (8.0 * (inptasksperf[..., 0] - inptasksperf[..., 1])
         + tasksobsids[..., 0].astype(jnp.float32)).reshape(nt, rows, lanes)
    pred = taskspredids.reshape(rows, lanes)

    rblk = 128
    grid = (rows // rblk,)
    trust = pl.pallas_call(
        _trust_body,
        grid=grid,
        in_specs=[
            pl.BlockSpec((nt, rblk, lanes), lambda r: (0, r, 0)),
            pl.BlockSpec((rblk, lanes), lambda r: (r, 0)),
        ],
        out_specs=pl.BlockSpec((rblk, lanes), lambda r: (r, 0)),
        out_shape=jax.ShapeDtypeStruct((rows, lanes), jnp.float32),
        compiler_params=pltpu.CompilerParams(
            allow_input_fusion=[True, True]),
    )(c, pred)

    return trust.reshape(nb, 1)


# speedup vs baseline: 1.0226x; 1.0226x over previous
import numpy as np
import jax
import jax.numpy as jnp
from jax import lax
from jax.experimental import pallas as pl
from jax.experimental.pallas import tpu as pltpu

_C = 128
_NID = 6


def _build_consts():
    np.random.seed(0)
    m = np.zeros((_C, _NID), dtype=np.float32)
    m[:, 1:_NID] = np.random.rand(_C, _NID - 1)
    colT = m.T

    X = np.zeros(_NID, dtype=np.uint32)
    for i in range(_NID):
        for y in range(32):
            if ((2 * y) >> i) & 1:
                X[i] |= np.uint32(1) << np.uint32(y)

    M = np.zeros(_NID, dtype=np.uint32)
    for j in range(1, _NID):
        for c in range(_C):
            pat = 0
            for i in range(_NID):
                if colT[i, c] >= colT[j, c]:
                    pat |= 1 << i
            M[j] |= np.uint32(1) << np.uint32(pat // 2)

    return ([int(v) for v in X.view(np.int32)],
            [int(v) for v in M.view(np.int32)])


_X32, _M32 = _build_consts()


def _select6(idx, consts):
    out = jnp.full(idx.shape, consts[0], dtype=jnp.int32)
    for i in range(1, _NID):
        out = jnp.where(idx == i, jnp.int32(consts[i]), out)
    return out


def _trust_body(c_ref, pred_ref, out_ref):
    nt = c_ref.shape[0]
    shp = c_ref.shape[1:]
    zero = jnp.zeros(shp, dtype=jnp.int32)
    neg1 = jnp.full(shp, -1, dtype=jnp.int32)

    def step(t, g):
        c = c_ref[t]
        s = c < -2.5
        f = c > 7.5
        idt = jnp.where(s, c + 8.0, jnp.where(f, c - 8.0, c))
        x = jnp.full(shp, _X32[0], dtype=jnp.int32)
        for i in range(1, _NID):
            x = jnp.where(idt == float(i), jnp.int32(_X32[i]), x)
        return (g | jnp.where(s, x, zero)) & jnp.where(f, x, neg1)

    g = lax.fori_loop(0, nt, step, zero, unroll=True)

    m = _select6(pred_ref[...], _M32)
    out_ref[...] = ((g & m) == m).astype(jnp.float32)


def kernel(inptasksobs, inptasksperf, inptaskspred, num_obs_tasks, tasksobsids, taskspredids):
    nt = tasksobsids.shape[0]
    nb = tasksobsids.shape[1]
    lanes = 128
    rows = nb // lanes

    c = (8.0 * (inptasksperf[..., 0] - inptasksperf[..., 1])
         + tasksobsids[..., 0].astype(jnp.float32)).reshape(nt, rows, lanes)
    pred = taskspredids.reshape(rows, lanes)

    rblk = 64
    grid = (rows // rblk,)
    trust = pl.pallas_call(
        _trust_body,
        grid=grid,
        in_specs=[
            pl.BlockSpec((nt, rblk, lanes), lambda r: (0, r, 0)),
            pl.BlockSpec((rblk, lanes), lambda r: (r, 0)),
        ],
        out_specs=pl.BlockSpec((rblk, lanes), lambda r: (r, 0)),
        out_shape=jax.ShapeDtypeStruct((rows, lanes), jnp.float32),
        compiler_params=pltpu.CompilerParams(
            allow_input_fusion=[True, True]),
    )(c, pred)

    return trust.reshape(nb, 1)
